# baseline (speedup 1.0000x reference)
import jax
import jax.numpy as jnp
from jax.experimental import pallas as pl
from jax.experimental.pallas import tpu as pltpu

_BLOCK = 14288
_PE_PAD = 32


def _fused_kernel(x_ref, ids_ref, w1_ref, b_ref, w2_ref, out_ref):
    x_blk = x_ref[...]
    ids = ids_ref[0, :, :]
    nrows = x_blk.shape[0]
    onehot_t = (jax.lax.broadcasted_iota(jnp.int32, (_PE_PAD, nrows), 0)
                == ids).astype(jnp.float32)
    h = jnp.dot(x_blk, w1_ref[...], preferred_element_type=jnp.float32)
    pos = jax.lax.dot_general(
        onehot_t, w2_ref[...],
        dimension_numbers=(((0,), (0,)), ((), ())),
        preferred_element_type=jnp.float32)
    out_ref[...] = h + pos + b_ref[0, :]


def kernel(x, batch, W, b, pe_table):
    n, dim_in = x.shape
    d_out = W.shape[1]
    dim_pe = pe_table.shape[1]
    d_full = d_out + dim_pe
    nb = -(-n // _BLOCK)
    ids_padded = jnp.zeros((nb * _BLOCK,), jnp.int32).at[:n].set(batch.astype(jnp.int32))
    ids3 = ids_padded.reshape(nb, 1, _BLOCK)
    w1 = jnp.zeros((dim_in, d_full), jnp.float32).at[:, :d_out].set(W)
    w2 = jnp.zeros((_PE_PAD, d_full), jnp.float32).at[:pe_table.shape[0], d_out:].set(pe_table)
    b2 = jnp.zeros((1, d_full), jnp.float32).at[0, :d_out].set(b)

    return pl.pallas_call(
        _fused_kernel,
        grid=(nb,),
        in_specs=[
            pl.BlockSpec((_BLOCK, dim_in), lambda i: (i, 0)),
            pl.BlockSpec((1, 1, _BLOCK), lambda i: (i, 0, 0)),
            pl.BlockSpec((dim_in, d_full), lambda i: (0, 0)),
            pl.BlockSpec((1, d_full), lambda i: (0, 0)),
            pl.BlockSpec((_PE_PAD, d_full), lambda i: (0, 0)),
        ],
        out_specs=pl.BlockSpec((_BLOCK, d_full), lambda i: (i, 0)),
        out_shape=jax.ShapeDtypeStruct((n, d_full), jnp.float32),
        compiler_params=pltpu.CompilerParams(
            dimension_semantics=("parallel",),
        ),
    )(x, ids3, w1, b2, w2)


# speedup vs baseline: 1.0655x; 1.0655x over previous
import jax
import jax.numpy as jnp
from jax.experimental import pallas as pl
from jax.experimental.pallas import tpu as pltpu

_BLOCK = 14288
_PE_PAD = 32


def _fused_kernel(x_ref, ids_ref, w_ref, b_ref, pe_ref, out_ref):
    x_blk = x_ref[...]
    d_out = out_ref.shape[1] - pe_ref.shape[1]
    h = jnp.dot(x_blk, w_ref[...], preferred_element_type=jnp.float32)
    out_ref[...] = h + b_ref[0, :]
    ids = ids_ref[0, :, :]
    onehot_t = (jax.lax.broadcasted_iota(jnp.int32, (_PE_PAD, x_blk.shape[0]), 0)
                == ids).astype(jnp.float32)
    pos = jax.lax.dot_general(
        onehot_t, pe_ref[...],
        dimension_numbers=(((0,), (0,)), ((), ())),
        preferred_element_type=jnp.float32)
    out_ref[:, d_out:] = pos


def kernel(x, batch, W, b, pe_table):
    n, dim_in = x.shape
    d_out = W.shape[1]
    dim_pe = pe_table.shape[1]
    nb = -(-n // _BLOCK)
    ids_padded = jnp.zeros((nb * _BLOCK,), jnp.int32).at[:n].set(batch.astype(jnp.int32))
    ids3 = ids_padded.reshape(nb, 1, _BLOCK)
    d_full = d_out + dim_pe
    w1 = jnp.zeros((dim_in, d_full), jnp.float32).at[:, :d_out].set(W)
    b2 = jnp.zeros((1, d_full), jnp.float32).at[0, :d_out].set(b)
    pe_pad = jnp.zeros((_PE_PAD, dim_pe), jnp.float32).at[:pe_table.shape[0]].set(pe_table)

    return pl.pallas_call(
        _fused_kernel,
        grid=(nb,),
        in_specs=[
            pl.BlockSpec((_BLOCK, dim_in), lambda i: (i, 0)),
            pl.BlockSpec((1, 1, _BLOCK), lambda i: (i, 0, 0)),
            pl.BlockSpec((dim_in, d_full), lambda i: (0, 0)),
            pl.BlockSpec((1, d_full), lambda i: (0, 0)),
            pl.BlockSpec((_PE_PAD, dim_pe), lambda i: (0, 0)),
        ],
        out_specs=pl.BlockSpec((_BLOCK, d_out + dim_pe), lambda i: (i, 0)),
        out_shape=jax.ShapeDtypeStruct((n, d_out + dim_pe), jnp.float32),
        compiler_params=pltpu.CompilerParams(
            dimension_semantics=("parallel",),
        ),
    )(x, ids3, w1, b2, pe_pad)


# speedup vs baseline: 1.0955x; 1.0281x over previous
import jax
import jax.numpy as jnp
from jax.experimental import pallas as pl
from jax.experimental.pallas import tpu as pltpu

_BLOCK = 14288
_PE_PAD = 32


def _fused_kernel(x_ref, ids_ref, w_ref, b_ref, pe_ref, out_ref):
    x_blk = x_ref[...]
    d_full = out_ref.shape[1]
    dim_pe = 16
    h = jnp.dot(x_blk, w_ref[...], preferred_element_type=jnp.float32)
    h = h + b_ref[0, :]
    ids = ids_ref[0, :, :]
    onehot_t = (jax.lax.broadcasted_iota(jnp.int32, (_PE_PAD, x_blk.shape[0]), 0)
                == ids).astype(jnp.float32)
    pos_rep = jax.lax.dot_general(
        onehot_t, pe_ref[...],
        dimension_numbers=(((0,), (0,)), ((), ())),
        preferred_element_type=jnp.float32)
    lane = jax.lax.broadcasted_iota(jnp.int32, h.shape, 1)
    out_ref[...] = jnp.where(lane >= d_full - dim_pe, pos_rep, h)


def kernel(x, batch, W, b, pe_table):
    n, dim_in = x.shape
    d_out = W.shape[1]
    dim_pe = pe_table.shape[1]
    d_full = d_out + dim_pe
    nb = -(-n // _BLOCK)
    ids_padded = jnp.zeros((nb * _BLOCK,), jnp.int32).at[:n].set(batch.astype(jnp.int32))
    ids3 = ids_padded.reshape(nb, 1, _BLOCK)
    w1 = jnp.zeros((dim_in, d_full), jnp.float32).at[:, :d_out].set(W)
    b2 = jnp.zeros((1, d_full), jnp.float32).at[0, :d_out].set(b)
    pe_rep = jnp.zeros((_PE_PAD, d_full), jnp.float32)
    pe_rep = pe_rep.at[:pe_table.shape[0]].set(jnp.tile(pe_table, (1, d_full // dim_pe)))

    return pl.pallas_call(
        _fused_kernel,
        grid=(nb,),
        in_specs=[
            pl.BlockSpec((_BLOCK, dim_in), lambda i: (i, 0)),
            pl.BlockSpec((1, 1, _BLOCK), lambda i: (i, 0, 0)),
            pl.BlockSpec((dim_in, d_full), lambda i: (0, 0)),
            pl.BlockSpec((1, d_full), lambda i: (0, 0)),
            pl.BlockSpec((_PE_PAD, d_full), lambda i: (0, 0)),
        ],
        out_specs=pl.BlockSpec((_BLOCK, d_full), lambda i: (i, 0)),
        out_shape=jax.ShapeDtypeStruct((n, d_full), jnp.float32),
        compiler_params=pltpu.CompilerParams(
            dimension_semantics=("parallel",),
        ),
    )(x, ids3, w1, b2, pe_rep)


# speedup vs baseline: 1.1380x; 1.0388x over previous
import jax
import jax.numpy as jnp
from jax.experimental import pallas as pl
from jax.experimental.pallas import tpu as pltpu

_BLOCK = 16672
_PE_PAD = 32


def _fused_kernel(x_ref, ids_ref, w_ref, b_ref, pe_ref, out_ref):
    x_blk = x_ref[...]
    d_out = w_ref.shape[1]
    h = jnp.dot(x_blk, w_ref[...], preferred_element_type=jnp.float32)
    out_ref[:, :d_out] = h + b_ref[0, :]
    ids = ids_ref[0, :, :]
    onehot_t = (jax.lax.broadcasted_iota(jnp.int32, (_PE_PAD, x_blk.shape[0]), 0)
                == ids).astype(jnp.float32)
    pos = jax.lax.dot_general(
        onehot_t, pe_ref[...],
        dimension_numbers=(((0,), (0,)), ((), ())),
        preferred_element_type=jnp.float32)
    out_ref[:, d_out:] = pos


def kernel(x, batch, W, b, pe_table):
    n, dim_in = x.shape
    d_out = W.shape[1]
    dim_pe = pe_table.shape[1]
    nb = -(-n // _BLOCK)
    ids_padded = jnp.zeros((nb * _BLOCK,), jnp.int32).at[:n].set(batch.astype(jnp.int32))
    ids3 = ids_padded.reshape(nb, 1, _BLOCK)
    b2 = b.reshape(1, d_out)
    pe_pad = jnp.zeros((_PE_PAD, dim_pe), jnp.float32).at[:pe_table.shape[0]].set(pe_table)

    return pl.pallas_call(
        _fused_kernel,
        grid=(nb,),
        in_specs=[
            pl.BlockSpec((_BLOCK, dim_in), lambda i: (i, 0)),
            pl.BlockSpec((1, 1, _BLOCK), lambda i: (i, 0, 0)),
            pl.BlockSpec((dim_in, d_out), lambda i: (0, 0)),
            pl.BlockSpec((1, d_out), lambda i: (0, 0)),
            pl.BlockSpec((_PE_PAD, dim_pe), lambda i: (0, 0)),
        ],
        out_specs=pl.BlockSpec((_BLOCK, d_out + dim_pe), lambda i: (i, 0)),
        out_shape=jax.ShapeDtypeStruct((n, d_out + dim_pe), jnp.float32),
        compiler_params=pltpu.CompilerParams(
            dimension_semantics=("parallel",),
        ),
    )(x, ids3, W, b2, pe_pad)
